# Initial kernel scaffold; baseline (speedup 1.0000x reference)
import jax
import jax.numpy as jnp
from jax.experimental import pallas as pl


def kernel(text, image, pos_table, text_pos_table, image_pos_table):
    raise NotImplementedError("write your pallas kernel here")


# speedup vs baseline: 1.5435x; 1.5435x over previous
import jax
import jax.numpy as jnp
from jax.experimental import pallas as pl


def _add_kernel(text_ref, image_ref, ttab_ref, itab_ref, tout_ref, iout_ref):
    tout_ref[...] = ttab_ref[...] + text_ref[...]
    iout_ref[...] = itab_ref[...] + image_ref[...]


def kernel(text, image, pos_table, text_pos_table, image_pos_table):
    del pos_table
    batch, seq_len = text.shape
    d_model = text_pos_table.shape[1]

    block_rows = 256
    grid = (seq_len // block_rows,)

    tout, iout = pl.pallas_call(
        _add_kernel,
        grid=grid,
        in_specs=[
            pl.BlockSpec((batch, d_model), lambda i: (0, 0)),
            pl.BlockSpec((batch, d_model), lambda i: (0, 0)),
            pl.BlockSpec((block_rows, d_model), lambda i: (i, 0)),
            pl.BlockSpec((block_rows, d_model), lambda i: (i, 0)),
        ],
        out_specs=[
            pl.BlockSpec((block_rows, d_model), lambda i: (i, 0)),
            pl.BlockSpec((block_rows, d_model), lambda i: (i, 0)),
        ],
        out_shape=[
            jax.ShapeDtypeStruct((seq_len, d_model), text.dtype),
            jax.ShapeDtypeStruct((seq_len, d_model), image.dtype),
        ],
    )(text, image, text_pos_table[:seq_len], image_pos_table[:seq_len])

    return (tout[None], iout[None])
